# baseline (speedup 1.0000x reference)
import functools

import jax
import jax.numpy as jnp
from jax import lax
from jax.experimental import pallas as pl
from jax.experimental.pallas import tpu as pltpu
from jax.experimental.pallas import tpu_sc as plsc

BATCH = 16384
EMB = 64
IDX_CHUNK = 128


def _sc_gather(user_idx2d, item_idx2d, user_emb, item_emb):
    info = plsc.get_sparse_core_info()
    nc, ns = info.num_cores, info.num_subcores
    nw = nc * ns
    rows_per_w = BATCH // nw
    chunks_per_w = rows_per_w // IDX_CHUNK

    mesh = plsc.VectorSubcoreMesh(core_axis_name="c", subcore_axis_name="s")

    @functools.partial(
        pl.kernel,
        mesh=mesh,
        compiler_params=pltpu.CompilerParams(use_tc_tiling_on_sc=False),
        out_type=[
            jax.ShapeDtypeStruct((BATCH, EMB), jnp.float32),
            jax.ShapeDtypeStruct((BATCH, EMB), jnp.float32),
        ],
        scratch_types=[
            pltpu.VMEM((chunks_per_w, IDX_CHUNK), jnp.int32),
            pltpu.VMEM((chunks_per_w, IDX_CHUNK), jnp.int32),
            pltpu.VMEM((rows_per_w, EMB), jnp.float32),
            pltpu.VMEM((rows_per_w, EMB), jnp.float32),
            pltpu.SemaphoreType.DMA,
        ],
    )
    def gather_k(uidx_hbm, iidx_hbm, uemb_hbm, iemb_hbm, out_u, out_i,
                 uidx_v, iidx_v, urows_v, irows_v, sem):
        wid = lax.axis_index("s") * nc + lax.axis_index("c")
        crow = wid * chunks_per_w
        pltpu.sync_copy(uidx_hbm.at[pl.ds(crow, chunks_per_w)], uidx_v)
        pltpu.sync_copy(iidx_hbm.at[pl.ds(crow, chunks_per_w)], iidx_v)
        cps = []
        for j in range(chunks_per_w):
            cps.append(pltpu.async_copy(
                uemb_hbm.at[uidx_v.at[j]],
                urows_v.at[pl.ds(j * IDX_CHUNK, IDX_CHUNK)], sem))
            cps.append(pltpu.async_copy(
                iemb_hbm.at[iidx_v.at[j]],
                irows_v.at[pl.ds(j * IDX_CHUNK, IDX_CHUNK)], sem))
        for cp in cps:
            cp.wait()
        base = wid * rows_per_w
        pltpu.sync_copy(urows_v, out_u.at[pl.ds(base, rows_per_w)])
        pltpu.sync_copy(irows_v, out_i.at[pl.ds(base, rows_per_w)])

    return gather_k(user_idx2d, item_idx2d, user_emb, item_emb)


def _mlp_block(u_ref, v_ref, w1u_ref, w1i_ref, b1_ref, w2_ref, b2_ref,
               w3_ref, b3_ref, wo_ref, bo_ref, out_ref):
    x = jnp.dot(u_ref[...], w1u_ref[...], preferred_element_type=jnp.float32)
    x += jnp.dot(v_ref[...], w1i_ref[...], preferred_element_type=jnp.float32)
    x = jnp.maximum(x + b1_ref[...], 0.0)
    x = jnp.dot(x, w2_ref[...], preferred_element_type=jnp.float32)
    x = jnp.maximum(x + b2_ref[...], 0.0)
    x = jnp.dot(x, w3_ref[...], preferred_element_type=jnp.float32)
    x = jnp.maximum(x + b3_ref[...], 0.0)
    logit = jnp.sum(x * wo_ref[...], axis=1) + bo_ref[0, 0]
    out_ref[...] = jax.nn.sigmoid(logit)


def kernel(user_indices, item_indices, user_emb, item_emb,
           W1, b1, W2, b2, W3, b3, Wo, bo):
    uidx2d = user_indices.reshape(BATCH // IDX_CHUNK, IDX_CHUNK)
    iidx2d = item_indices.reshape(BATCH // IDX_CHUNK, IDX_CHUNK)
    u_rows, i_rows = _sc_gather(uidx2d, iidx2d, user_emb, item_emb)

    w1u = W1[:, :EMB].T
    w1i = W1[:, EMB:].T
    w2t = W2.T
    w3t = W3.T
    b1r = b1.reshape(1, -1)
    b2r = b2.reshape(1, -1)
    b3r = b3.reshape(1, -1)
    wor = Wo.reshape(1, -1)
    bor = bo.reshape(1, 1)

    bb = 2048
    grid = (BATCH // bb,)
    full = lambda i: (0, 0)
    out = pl.pallas_call(
        _mlp_block,
        grid=grid,
        in_specs=[
            pl.BlockSpec((bb, EMB), lambda i: (i, 0)),
            pl.BlockSpec((bb, EMB), lambda i: (i, 0)),
            pl.BlockSpec(w1u.shape, full),
            pl.BlockSpec(w1i.shape, full),
            pl.BlockSpec(b1r.shape, full),
            pl.BlockSpec(w2t.shape, full),
            pl.BlockSpec(b2r.shape, full),
            pl.BlockSpec(w3t.shape, full),
            pl.BlockSpec(b3r.shape, full),
            pl.BlockSpec(wor.shape, full),
            pl.BlockSpec(bor.shape, full),
        ],
        out_specs=pl.BlockSpec((bb,), lambda i: (i,)),
        out_shape=jax.ShapeDtypeStruct((BATCH,), jnp.float32),
    )(u_rows, i_rows, w1u, w1i, b1r, w2t, b2r, w3t, b3r, wor, bor)
    return out
